# baseline (speedup 1.0000x reference)
import functools

import jax
import jax.numpy as jnp
from jax import lax
from jax.experimental import pallas as pl
from jax.experimental.pallas import tpu as pltpu
from jax.experimental.pallas import tpu_sc as plsc

_K = 512
_D = 64
_NC = 2
_NS = 16
_NW = _NC * _NS
_L = 16


def _argmin_body(z_ref, w_ref, idx_ref):
    x = z_ref[0]
    w = w_ref[...]
    hw = x.shape[1]

    wn = jnp.sum(w * w, axis=1, keepdims=True)
    zn = jnp.sum(x * x, axis=0, keepdims=True)
    scores = lax.dot_general(
        w, x, dimension_numbers=(((1,), (0,)), ((), ())),
        preferred_element_type=jnp.float32)
    d = (zn + wn) - 2.0 * scores

    m = jnp.min(d, axis=0, keepdims=True)
    rows = lax.broadcasted_iota(jnp.int32, (_K, hw), 0)
    idx_ref[0, 0] = jnp.min(jnp.where(d <= m, rows, _K), axis=0)


def _sc_gather_body(wt_hbm, idx_hbm, out_hbm, wt_v, idx_v, out_v):
    nb = idx_hbm.shape[0]
    per_w = nb // _NW
    wid = lax.axis_index("s") * _NC + lax.axis_index("c")
    hw = idx_hbm.shape[1]
    pltpu.sync_copy(wt_hbm, wt_v)
    for j in range(per_w):
        b = wid * per_w + j
        pltpu.sync_copy(idx_hbm.at[b], idx_v)

        def grp(g, carry):
            vidx = idx_v[pl.ds(g * _L, _L)]
            for c in range(_D):
                row = plsc.load_gather(
                    wt_v, [jnp.full((_L,), c, jnp.int32), vidx])
                out_v[c, pl.ds(g * _L, _L)] = row
            return carry

        lax.fori_loop(0, hw // _L, grp, 0)
        pltpu.sync_copy(out_v, out_hbm.at[b])


def kernel(z_e, weight):
    B, C, H, W = z_e.shape
    hw = H * W
    z = z_e.reshape(B, C, hw)
    wt = jnp.transpose(weight, (1, 0))

    idx = pl.pallas_call(
        _argmin_body,
        grid=(B,),
        in_specs=[
            pl.BlockSpec((1, C, hw), lambda b: (b, 0, 0)),
            pl.BlockSpec((_K, _D), lambda b: (0, 0)),
        ],
        out_specs=pl.BlockSpec((1, 1, hw), lambda b: (b, 0, 0)),
        out_shape=jax.ShapeDtypeStruct((B, 1, hw), jnp.int32),
    )(z, weight)

    sc_gather = functools.partial(
        pl.kernel,
        out_type=jax.ShapeDtypeStruct((B, _D, hw), jnp.float32),
        mesh=plsc.VectorSubcoreMesh(core_axis_name="c", subcore_axis_name="s"),
        scratch_types=[
            pltpu.VMEM((_D, _K), jnp.float32),
            pltpu.VMEM((hw,), jnp.int32),
            pltpu.VMEM((_D, hw), jnp.float32),
        ],
        compiler_params=pltpu.CompilerParams(
            use_tc_tiling_on_sc=False, needs_layout_passes=False),
    )(_sc_gather_body)

    zq = sc_gather(wt, idx.reshape(B, hw))
    return zq.reshape(B, C, H, W)


# speedup vs baseline: 1.0470x; 1.0470x over previous
import functools

import jax
import jax.numpy as jnp
from jax import lax
from jax.experimental import pallas as pl
from jax.experimental.pallas import tpu as pltpu
from jax.experimental.pallas import tpu_sc as plsc

_K = 512
_D = 64
_NC = 2
_NS = 16
_NW = _NC * _NS
_L = 16


def _argmin_body(z_ref, w_ref, idx_ref):
    x = z_ref[0]
    w = w_ref[...]
    hw = x.shape[1]

    wn = jnp.sum(w * w, axis=1, keepdims=True)
    zn = jnp.sum(x * x, axis=0, keepdims=True)
    scores = lax.dot_general(
        w, x, dimension_numbers=(((1,), (0,)), ((), ())),
        preferred_element_type=jnp.float32)
    d = (zn + wn) - 2.0 * scores

    m = jnp.min(d, axis=0, keepdims=True)
    rows = lax.broadcasted_iota(jnp.int32, (_K, hw), 0)
    idx_ref[0, 0] = jnp.min(jnp.where(d <= m, rows, _K), axis=0)


def _sc_gather_body(wt_hbm, idx_hbm, out_hbm, wt_v, idx_v, out_v):
    nb = idx_hbm.shape[0]
    per_w = nb // _NW
    wid = lax.axis_index("s") * _NC + lax.axis_index("c")
    hw = idx_hbm.shape[1]
    pltpu.sync_copy(wt_hbm, wt_v)
    for j in range(per_w):
        b = wid * per_w + j
        pltpu.sync_copy(idx_hbm.at[b], idx_v)

        def grp(g, carry):
            vidx = idx_v[pl.ds(g * _L, _L)]
            for c in range(_D):
                row = plsc.load_gather(
                    wt_v, [jnp.full((_L,), c, jnp.int32), vidx])
                out_v[c, pl.ds(g * _L, _L)] = row
            return carry

        lax.fori_loop(0, hw // _L, grp, 0)
        pltpu.sync_copy(out_v, out_hbm.at[b])


def kernel(z_e, weight):
    B, C, H, W = z_e.shape
    hw = H * W
    z = z_e.reshape(B, C, hw)
    wt = jnp.transpose(weight, (1, 0))

    idx = pl.pallas_call(
        _argmin_body,
        grid=(B,),
        in_specs=[
            pl.BlockSpec((1, C, hw), lambda b: (b, 0, 0)),
            pl.BlockSpec((_K, _D), lambda b: (0, 0)),
        ],
        out_specs=pl.BlockSpec((1, 1, hw), lambda b: (b, 0, 0)),
        out_shape=jax.ShapeDtypeStruct((B, 1, hw), jnp.int32),
    )(z, weight)

    sc_gather = functools.partial(
        pl.kernel,
        out_type=jax.ShapeDtypeStruct((B, _D, hw), jnp.float32),
        mesh=plsc.VectorSubcoreMesh(core_axis_name="c", subcore_axis_name="s"),
        scratch_types=[
            pltpu.VMEM((_D, _K), jnp.float32),
            pltpu.VMEM((hw,), jnp.int32),
            pltpu.VMEM((_D, hw), jnp.float32),
        ],
        compiler_params=pltpu.CompilerParams(needs_layout_passes=False),
    )(_sc_gather_body)

    zq = sc_gather(wt, idx.reshape(B, hw))
    return zq.reshape(B, C, H, W)


# speedup vs baseline: 1.1151x; 1.0651x over previous
import functools

import jax
import jax.numpy as jnp
from jax import lax
from jax.experimental import pallas as pl
from jax.experimental.pallas import tpu as pltpu
from jax.experimental.pallas import tpu_sc as plsc

_K = 512
_D = 64
_NC = 2
_NS = 16
_NW = _NC * _NS
_L = 16
_CB = 16


def _argmin_body(z_ref, w_ref, idx_ref):
    x = z_ref[0]
    w = w_ref[...]
    hw = x.shape[1]

    wn = jnp.sum(w * w, axis=1, keepdims=True)
    zn = jnp.sum(x * x, axis=0, keepdims=True)
    scores = lax.dot_general(
        w, x, dimension_numbers=(((1,), (0,)), ((), ())),
        preferred_element_type=jnp.float32)
    d = (zn + wn) - 2.0 * scores

    m = jnp.min(d, axis=0, keepdims=True)
    rows = lax.broadcasted_iota(jnp.int32, (_K, hw), 0)
    idx_ref[0, 0] = jnp.min(jnp.where(d <= m, rows, _K), axis=0)


def _sc_gather_body(wt_hbm, idx_hbm, out_hbm, wt_v, idx_v, out_v):
    wid = lax.axis_index("s") * _NC + lax.axis_index("c")
    P = idx_v.shape[0]
    hw = out_hbm.shape[2]
    start = wid * P
    b = start // hw
    off = start % hw

    pltpu.sync_copy(wt_hbm, wt_v)
    pltpu.sync_copy(idx_hbm.at[pl.ds(start, P)], idx_v)

    def grp(g, carry):
        vidx = idx_v[pl.ds(g * _L, _L)]
        for c in range(_D):
            row = plsc.load_gather(
                wt_v, [jnp.full((_L,), c, jnp.int32), vidx])
            out_v[c, pl.ds(g * _L, _L)] = row
        return carry

    lax.fori_loop(0, P // _L, grp, 0)
    pltpu.sync_copy(out_v, out_hbm.at[b, :, pl.ds(off, P)])


def kernel(z_e, weight):
    B, C, H, W = z_e.shape
    hw = H * W
    z = z_e.reshape(B, C, hw)
    wt = jnp.transpose(weight, (1, 0))
    P = _CB * hw // _NW

    sc_gather = functools.partial(
        pl.kernel,
        out_type=jax.ShapeDtypeStruct((_CB, _D, hw), jnp.float32),
        mesh=plsc.VectorSubcoreMesh(core_axis_name="c", subcore_axis_name="s"),
        scratch_types=[
            pltpu.VMEM((_D, _K), jnp.float32),
            pltpu.VMEM((P,), jnp.int32),
            pltpu.VMEM((_D, P), jnp.float32),
        ],
        compiler_params=pltpu.CompilerParams(needs_layout_passes=False),
    )(_sc_gather_body)

    outs = []
    for g in range(B // _CB):
        idx = pl.pallas_call(
            _argmin_body,
            grid=(_CB,),
            in_specs=[
                pl.BlockSpec((1, C, hw), lambda b, g=g: (g * _CB + b, 0, 0)),
                pl.BlockSpec((_K, _D), lambda b: (0, 0)),
            ],
            out_specs=pl.BlockSpec((1, 1, hw), lambda b: (b, 0, 0)),
            out_shape=jax.ShapeDtypeStruct((_CB, 1, hw), jnp.int32),
        )(z, weight)
        outs.append(sc_gather(wt, idx.reshape(_CB * hw)))

    zq = jnp.concatenate(outs, axis=0)
    return zq.reshape(B, C, H, W)


# speedup vs baseline: 1.5786x; 1.4156x over previous
import jax
import jax.numpy as jnp
from jax import lax
from jax.experimental import pallas as pl

_K = 512
_D = 64
_BB = 4


def _vq_body(z_ref, w_ref, wt_ref, out_ref):
    w = w_ref[...]
    wt = wt_ref[...]
    wn = jnp.sum(w * w, axis=1, keepdims=True)
    for j in range(_BB):
        x = z_ref[j]
        hw = x.shape[1]
        zn = jnp.sum(x * x, axis=0, keepdims=True)
        scores = lax.dot_general(
            w, x, dimension_numbers=(((1,), (0,)), ((), ())),
            preferred_element_type=jnp.float32)
        d = (zn + wn) - 2.0 * scores

        m = jnp.min(d, axis=0, keepdims=True)
        rows = lax.broadcasted_iota(jnp.int32, (_K, hw), 0)
        idx = jnp.min(jnp.where(d <= m, rows, _K), axis=0)

        onehot = (rows == idx[None, :]).astype(jnp.float32)
        out_ref[j] = lax.dot_general(
            wt, onehot, dimension_numbers=(((1,), (0,)), ((), ())),
            preferred_element_type=jnp.float32)


def kernel(z_e, weight):
    B, C, H, W = z_e.shape
    hw = H * W
    z = z_e.reshape(B, C, hw)
    wt = jnp.transpose(weight, (1, 0))

    out = pl.pallas_call(
        _vq_body,
        grid=(B // _BB,),
        in_specs=[
            pl.BlockSpec((_BB, C, hw), lambda b: (b, 0, 0)),
            pl.BlockSpec((_K, _D), lambda b: (0, 0)),
            pl.BlockSpec((_D, _K), lambda b: (0, 0)),
        ],
        out_specs=pl.BlockSpec((_BB, C, hw), lambda b: (b, 0, 0)),
        out_shape=jax.ShapeDtypeStruct((B, C, hw), jnp.float32),
    )(z, weight, wt)
    return out.reshape(B, C, H, W)
